# baseline (speedup 1.0000x reference)
import jax
import jax.numpy as jnp
from jax.experimental import pallas as pl
from jax.experimental.pallas import tpu as pltpu

_NCHUNK = 8


def _dma_copy(t_ref, o_ref, sems):
    rows = t_ref.shape[0]
    c = rows // _NCHUNK
    for i in range(_NCHUNK):
        pltpu.make_async_copy(
            t_ref.at[pl.ds(i * c, c)], o_ref.at[pl.ds(i * c, c)], sems.at[i]
        ).start()
    for i in range(_NCHUNK):
        pltpu.make_async_copy(
            t_ref.at[pl.ds(i * c, c)], o_ref.at[pl.ds(i * c, c)], sems.at[i]
        ).wait()


def kernel(x, table):
    del x
    T, E = table.shape
    out = pl.pallas_call(
        _dma_copy,
        in_specs=[pl.BlockSpec(memory_space=pl.ANY)],
        out_specs=pl.BlockSpec(memory_space=pl.ANY),
        out_shape=jax.ShapeDtypeStruct((T, E), table.dtype),
        scratch_shapes=[pltpu.SemaphoreType.DMA((_NCHUNK,))],
    )(table)
    return out[None, :, :]


# speedup vs baseline: 41.2173x; 41.2173x over previous
import jax
import jax.numpy as jnp
from jax.experimental import pallas as pl

_ROWS = 256


def _copy_block(t_ref, o_ref):
    o_ref[...] = t_ref[...]


def kernel(x, table):
    del x
    T, E = table.shape
    out = pl.pallas_call(
        _copy_block,
        grid=(T // _ROWS,),
        in_specs=[pl.BlockSpec((_ROWS, E), lambda i: (i, 0))],
        out_specs=pl.BlockSpec((_ROWS, E), lambda i: (i, 0)),
        out_shape=jax.ShapeDtypeStruct((T, E), table.dtype),
    )(table)
    return out[None, :, :]


# speedup vs baseline: 49.1742x; 1.1930x over previous
import jax
import jax.numpy as jnp
from jax.experimental import pallas as pl

_ROWS = 1024


def _copy_block(t_ref, o_ref):
    o_ref[...] = t_ref[...]


def kernel(x, table):
    del x
    T, E = table.shape
    out = pl.pallas_call(
        _copy_block,
        grid=(T // _ROWS,),
        in_specs=[pl.BlockSpec((_ROWS, E), lambda i: (i, 0))],
        out_specs=pl.BlockSpec((_ROWS, E), lambda i: (i, 0)),
        out_shape=jax.ShapeDtypeStruct((T, E), table.dtype),
    )(table)
    return out[None, :, :]
